# baseline (speedup 1.0000x reference)
import functools
import math

import jax
import jax.numpy as jnp
from jax.experimental import pallas as pl
from jax.experimental.pallas import tpu as pltpu

_INTERPRET = False

_NP = 7


def _gelu(x):
    return 0.5 * x * (1.0 + jax.lax.erf(x * (1.0 / math.sqrt(2.0))))


def _gelu_t(x):
    c = math.sqrt(2.0 / math.pi)
    return 0.5 * x * (1.0 + jnp.tanh(c * (x + 0.044715 * x * x * x)))


def _dotc(w, x):
    return jax.lax.dot_general(
        w, x, (((0,), (0,)), ((), ())), preferred_element_type=jnp.float32
    )


def _dot(a, b):
    return jnp.dot(a, b, preferred_element_type=jnp.float32)


def _fused_body(x_ref, few_ref, feb_ref,
                w1_ref, b1_ref, cw1_ref, cb1_ref, cw2_ref, cb2_ref,
                w2_ref, b2_ref, w3_ref, b3_ref,
                sw_ref, sb_ref, ew_ref, eb_ref,
                out_ref,
                ft_ref, acc_ref, ti_ref, tw_ref,
                *, nB, TN, inv_p):
    b = pl.program_id(0)
    j = pl.program_id(1)
    C = x_ref.shape[1]
    hidden = out_ref.shape[1]
    TH = x_ref.shape[2]
    W = x_ref.shape[3]

    @pl.when((b > 0) & (j == 0))
    def _():
        g = acc_ref[...] * inv_p
        z = _gelu(_dot(g, w1_ref[...]) + b1_ref[...])
        a = _gelu(_dot(z, cw1_ref[...]) + cb1_ref[...])
        a = _dot(a, cw2_ref[...]) + cb2_ref[...]
        z = z * jax.nn.sigmoid(2.0 * a)
        z = _gelu(_dot(z, w2_ref[...]) + b2_ref[...])
        s = _dot(z, w3_ref[...]) + b3_ref[...]
        idx = jax.lax.broadcasted_iota(jnp.int32, s.shape, 1)
        m1 = jnp.max(s)
        i1 = jnp.min(jnp.where(s >= m1, idx, 127))
        s2 = jnp.where(idx == i1, -jnp.inf, s)
        m2 = jnp.max(s2)
        i2 = jnp.min(jnp.where(s2 >= m2, idx, 127))
        d = jnp.exp((m2 - m1) * 0.5)
        ti_ref[0] = i1
        ti_ref[1] = i2
        tw_ref[0] = 1.0 / (1.0 + d)
        tw_ref[1] = d / (1.0 + d)

    @pl.when(b < nB)
    def _():
        xb = x_ref[0].reshape(C, TN)
        ft = _gelu(_dotc(few_ref[...], xb) + feb_ref[0][:, None])
        ft_ref[b % 2, :, pl.ds(j * TN, TN)] = ft.astype(jnp.bfloat16)
        s = jnp.sum(ft, axis=1)[None, :]

        @pl.when(j == 0)
        def _():
            acc_ref[...] = s

        @pl.when(j > 0)
        def _():
            acc_ref[...] = acc_ref[...] + s

    @pl.when(b > 0)
    def _():
        ftb = ft_ref[(b + 1) % 2, :, pl.ds(j * TN, TN)]
        acc = 0.5 * _gelu_t(_dotc(sw_ref[0], ftb) + sb_ref[0, 0][:, None])
        acc = acc + 0.5 * _gelu_t(_dotc(sw_ref[1], ftb) + sb_ref[1, 0][:, None])
        i0 = ti_ref[0]
        i1 = ti_ref[1]
        acc = acc + tw_ref[0] * _gelu_t(_dotc(ew_ref[i0], ftb)
                                        + eb_ref[i0, 0][:, None])
        acc = acc + tw_ref[1] * _gelu_t(_dotc(ew_ref[i1], ftb)
                                        + eb_ref[i1, 0][:, None])
        out_ref[0] = acc.reshape(hidden, TH, W)


def kernel(x, fe_w, fe_b, g_w1, g_b1, bn1_g, bn1_b, ca_w1, ca_b1, ca_w2, ca_b2,
           g_w2, g_b2, bn2_g, bn2_b, g_w3, g_b3, shared_w, shared_b,
           expert_w, expert_b):
    B, C, H, W = x.shape
    P = H * W
    hidden = fe_w.shape[1]
    h2 = g_w1.shape[1]
    E = expert_w.shape[0]
    nP = _NP
    TN = P // nP
    TH = H // nP
    f32 = jnp.float32
    bf16 = jnp.bfloat16

    feb2 = fe_b.reshape(1, hidden)

    c = 1.0 / math.sqrt(1.0 + 1e-5)
    s1 = bn1_g * c
    w1f = g_w1 * s1[None, :]
    b1f = (g_b1 * s1 + bn1_b).reshape(1, h2)
    s2 = bn2_g * c
    w2f = g_w2 * s2[None, :]
    b2f = (g_b2 * s2 + bn2_b).reshape(1, hidden)

    red = ca_w1.shape[1]
    cw1p = jnp.pad(ca_w1, ((0, 0), (0, 128 - red)))
    cb1p = jnp.pad(ca_b1, (0, 128 - red)).reshape(1, 128)
    cw2p = jnp.pad(ca_w2, ((0, 128 - red), (0, 0)))
    cb2r = ca_b2.reshape(1, h2)
    w3p = jnp.pad(g_w3, ((0, 0), (0, 128 - E)))
    b3p = jnp.pad(g_b3, (0, 128 - E), constant_values=-1e9).reshape(1, 128)

    swb = shared_w.astype(bf16)
    ewb = expert_w.astype(bf16)
    eb3 = expert_b.reshape(E, 1, hidden)
    sb3 = shared_b.reshape(2, 1, hidden)

    cm = lambda b, j: (0, 0)
    cm3 = lambda b, j: (0, 0, 0)
    nB = B

    out = pl.pallas_call(
        functools.partial(_fused_body, nB=nB, TN=TN, inv_p=1.0 / P),
        grid=(B + 1, nP),
        in_specs=[
            pl.BlockSpec((1, C, TH, W),
                         lambda b, j: (jnp.minimum(b, nB - 1), 0,
                                       jnp.where(b < nB, j, nP - 1), 0)),
            pl.BlockSpec((C, hidden), cm),
            pl.BlockSpec((1, hidden), cm),
            pl.BlockSpec((hidden, h2), cm),
            pl.BlockSpec((1, h2), cm),
            pl.BlockSpec((h2, 128), cm),
            pl.BlockSpec((1, 128), cm),
            pl.BlockSpec((128, h2), cm),
            pl.BlockSpec((1, h2), cm),
            pl.BlockSpec((h2, hidden), cm),
            pl.BlockSpec((1, hidden), cm),
            pl.BlockSpec((hidden, 128), cm),
            pl.BlockSpec((1, 128), cm),
            pl.BlockSpec((2, hidden, hidden), cm3),
            pl.BlockSpec((2, 1, hidden), cm3),
            pl.BlockSpec((E, hidden, hidden), cm3),
            pl.BlockSpec((E, 1, hidden), cm3),
        ],
        out_specs=pl.BlockSpec(
            (1, hidden, TH, W),
            lambda b, j: (jnp.maximum(b, 1) - 1, 0, jnp.where(b > 0, j, 0), 0)),
        out_shape=jax.ShapeDtypeStruct((B, hidden, H, W), f32),
        scratch_shapes=[
            pltpu.VMEM((2, hidden, P), bf16),
            pltpu.VMEM((1, hidden), f32),
            pltpu.SMEM((2,), jnp.int32),
            pltpu.SMEM((2,), f32),
        ],
        interpret=_INTERPRET,
    )(x, fe_w, feb2, w1f, b1f, cw1p, cb1p, cw2p, cb2r, w2f, b2f, w3p, b3p,
      swb, sb3, ewb, eb3)

    return out


# speedup vs baseline: 1.1783x; 1.1783x over previous
import functools
import math

import jax
import jax.numpy as jnp
from jax.experimental import pallas as pl
from jax.experimental.pallas import tpu as pltpu

_INTERPRET = False

_NP = 7


def _gelu(x):
    return 0.5 * x * (1.0 + jax.lax.erf(x * (1.0 / math.sqrt(2.0))))


def _dotc(w, x):
    return jax.lax.dot_general(
        w, x, (((0,), (0,)), ((), ())), preferred_element_type=jnp.float32
    )


def _dot(a, b):
    return jnp.dot(a, b, preferred_element_type=jnp.float32)


def _fused_body(x_ref, few_ref, feb_ref,
                w1_ref, b1_ref, cw1_ref, cb1_ref, cw2_ref, cb2_ref,
                w2_ref, b2_ref, w3_ref, b3_ref,
                sw_ref, sb_ref, ew_ref, eb_ref,
                out_ref,
                ft_ref, acc_ref, ti_ref, tw_ref,
                *, nB, TN, inv_p):
    b = pl.program_id(0)
    j = pl.program_id(1)
    C = x_ref.shape[1]
    hidden = out_ref.shape[1]
    TH = x_ref.shape[2]
    W = x_ref.shape[3]

    @pl.when((b > 0) & (j == 0))
    def _():
        g = acc_ref[...] * inv_p
        z = _gelu(_dot(g, w1_ref[...]) + b1_ref[...])
        a = _gelu(_dot(z, cw1_ref[...]) + cb1_ref[...])
        a = _dot(a, cw2_ref[...]) + cb2_ref[...]
        z = z * jax.nn.sigmoid(2.0 * a)
        z = _gelu(_dot(z, w2_ref[...]) + b2_ref[...])
        s = _dot(z, w3_ref[...]) + b3_ref[...]
        idx = jax.lax.broadcasted_iota(jnp.int32, s.shape, 1)
        m1 = jnp.max(s)
        i1 = jnp.min(jnp.where(s >= m1, idx, 127))
        s2 = jnp.where(idx == i1, -jnp.inf, s)
        m2 = jnp.max(s2)
        i2 = jnp.min(jnp.where(s2 >= m2, idx, 127))
        d = jnp.exp((m2 - m1) * 0.5)
        ti_ref[0] = i1
        ti_ref[1] = i2
        tw_ref[0] = 1.0 / (1.0 + d)
        tw_ref[1] = d / (1.0 + d)

    @pl.when(b < nB)
    def _():
        xb = x_ref[0].reshape(C, TN)
        ft = _gelu(_dotc(few_ref[...], xb) + feb_ref[0][:, None])
        ft_ref[b % 2, :, pl.ds(j * TN, TN)] = ft.astype(jnp.bfloat16)
        s = jnp.sum(ft, axis=1)[None, :]

        @pl.when(j == 0)
        def _():
            acc_ref[...] = s

        @pl.when(j > 0)
        def _():
            acc_ref[...] = acc_ref[...] + s

    @pl.when(b > 0)
    def _():
        ftb = ft_ref[(b + 1) % 2, :, pl.ds(j * TN, TN)]
        acc = 0.5 * _gelu(_dotc(sw_ref[0], ftb) + sb_ref[0, 0][:, None])
        acc = acc + 0.5 * _gelu(_dotc(sw_ref[1], ftb) + sb_ref[1, 0][:, None])
        i0 = ti_ref[0]
        i1 = ti_ref[1]
        acc = acc + tw_ref[0] * _gelu(_dotc(ew_ref[i0], ftb)
                                      + eb_ref[i0, 0][:, None])
        acc = acc + tw_ref[1] * _gelu(_dotc(ew_ref[i1], ftb)
                                      + eb_ref[i1, 0][:, None])
        out_ref[0] = acc.reshape(hidden, TH, W)


def kernel(x, fe_w, fe_b, g_w1, g_b1, bn1_g, bn1_b, ca_w1, ca_b1, ca_w2, ca_b2,
           g_w2, g_b2, bn2_g, bn2_b, g_w3, g_b3, shared_w, shared_b,
           expert_w, expert_b):
    B, C, H, W = x.shape
    P = H * W
    hidden = fe_w.shape[1]
    h2 = g_w1.shape[1]
    E = expert_w.shape[0]
    nP = _NP
    TN = P // nP
    TH = H // nP
    f32 = jnp.float32
    bf16 = jnp.bfloat16

    feb2 = fe_b.reshape(1, hidden)

    c = 1.0 / math.sqrt(1.0 + 1e-5)
    s1 = bn1_g * c
    w1f = g_w1 * s1[None, :]
    b1f = (g_b1 * s1 + bn1_b).reshape(1, h2)
    s2 = bn2_g * c
    w2f = g_w2 * s2[None, :]
    b2f = (g_b2 * s2 + bn2_b).reshape(1, hidden)

    red = ca_w1.shape[1]
    cw1p = jnp.pad(ca_w1, ((0, 0), (0, 128 - red)))
    cb1p = jnp.pad(ca_b1, (0, 128 - red)).reshape(1, 128)
    cw2p = jnp.pad(ca_w2, ((0, 128 - red), (0, 0)))
    cb2r = ca_b2.reshape(1, h2)
    w3p = jnp.pad(g_w3, ((0, 0), (0, 128 - E)))
    b3p = jnp.pad(g_b3, (0, 128 - E), constant_values=-1e9).reshape(1, 128)

    swb = shared_w.astype(bf16)
    ewb = expert_w.astype(bf16)
    eb3 = expert_b.reshape(E, 1, hidden)
    sb3 = shared_b.reshape(2, 1, hidden)

    cm = lambda b, j: (0, 0)
    cm3 = lambda b, j: (0, 0, 0)
    nB = B

    out = pl.pallas_call(
        functools.partial(_fused_body, nB=nB, TN=TN, inv_p=1.0 / P),
        grid=(B + 1, nP),
        in_specs=[
            pl.BlockSpec((1, C, TH, W),
                         lambda b, j: (jnp.minimum(b, nB - 1), 0,
                                       jnp.where(b < nB, j, nP - 1), 0)),
            pl.BlockSpec((C, hidden), cm),
            pl.BlockSpec((1, hidden), cm),
            pl.BlockSpec((hidden, h2), cm),
            pl.BlockSpec((1, h2), cm),
            pl.BlockSpec((h2, 128), cm),
            pl.BlockSpec((1, 128), cm),
            pl.BlockSpec((128, h2), cm),
            pl.BlockSpec((1, h2), cm),
            pl.BlockSpec((h2, hidden), cm),
            pl.BlockSpec((1, hidden), cm),
            pl.BlockSpec((hidden, 128), cm),
            pl.BlockSpec((1, 128), cm),
            pl.BlockSpec((2, hidden, hidden), cm3),
            pl.BlockSpec((2, 1, hidden), cm3),
            pl.BlockSpec((E, hidden, hidden), cm3),
            pl.BlockSpec((E, 1, hidden), cm3),
        ],
        out_specs=pl.BlockSpec(
            (1, hidden, TH, W),
            lambda b, j: (jnp.maximum(b, 1) - 1, 0, jnp.where(b > 0, j, 0), 0)),
        out_shape=jax.ShapeDtypeStruct((B, hidden, H, W), f32),
        scratch_shapes=[
            pltpu.VMEM((2, hidden, P), bf16),
            pltpu.VMEM((1, hidden), f32),
            pltpu.SMEM((2,), jnp.int32),
            pltpu.SMEM((2,), f32),
        ],
        interpret=_INTERPRET,
    )(x, fe_w, feb2, w1f, b1f, cw1p, cb1p, cw2p, cb2r, w2f, b2f, w3p, b3p,
      swb, sb3, ewb, eb3)

    return out
